# baseline (speedup 1.0000x reference)
import jax
import jax.numpy as jnp
from jax import lax
from jax.experimental import pallas as pl

_SCORE_THRESH = 0.25
_NMS_THRESH = 0.45
_DETECTIONS = 300
_K_PRE = 1000
_MAX_SIZE = 4096.0
_IM = 640.0
_NC = 80
_N0, _N1, _N2 = 19200, 4800, 1200
_NB = _N0 + _N1 + _N2
_KPAD = 1024


def _score_kernel(p_ref, out_ref, mb_ref):
    s = jax.nn.sigmoid(p_ref[0][:, 4:85])
    obj = s[:, 0:1]
    cls = s[:, 1:81]
    gate = (obj > _SCORE_THRESH).astype(jnp.float32)
    sc = (obj * cls) * gate
    sm = jnp.where(sc > _SCORE_THRESH, sc, -1.0)
    out_ref[0] = sm
    mb_ref[0] = jnp.max(sm, axis=1, keepdims=True)


def _score_call(p, chunk):
    B, N, _ = p.shape
    return pl.pallas_call(
        _score_kernel,
        grid=(B, N // chunk),
        in_specs=[pl.BlockSpec((1, chunk, 85), lambda i, j: (i, j, 0))],
        out_specs=[
            pl.BlockSpec((1, chunk, _NC), lambda i, j: (i, j, 0)),
            pl.BlockSpec((1, chunk, 1), lambda i, j: (i, j, 0)),
        ],
        out_shape=[
            jax.ShapeDtypeStruct((B, N, _NC), jnp.float32),
            jax.ShapeDtypeStruct((B, N, 1), jnp.float32),
        ],
    )(p)


def _box_kernel(t_ref, out_ref):
    s = jax.nn.sigmoid(t_ref[0])
    b = lax.broadcasted_iota(jnp.int32, (1, _NB), 1)
    l1 = b >= _N0
    l2 = b >= (_N0 + _N1)
    rel = b - jnp.where(l2, _N0 + _N1, jnp.where(l1, _N0, 0))
    y = jnp.where(l2, rel // 60, jnp.where(l1, rel // 120, rel // 240))
    q = rel // 3
    x = jnp.where(l2, q % 20, jnp.where(l1, q % 40, q % 80))
    a = rel - q * 3
    stride = jnp.where(l2, 32.0, jnp.where(l1, 16.0, 8.0))

    def sel3(v0, v1, v2):
        return jnp.where(a == 0, v0, jnp.where(a == 1, v1, v2))

    aw = jnp.where(l2, sel3(116.0, 156.0, 373.0),
                   jnp.where(l1, sel3(30.0, 62.0, 59.0), sel3(10.0, 16.0, 33.0)))
    ah = jnp.where(l2, sel3(90.0, 198.0, 326.0),
                   jnp.where(l1, sel3(61.0, 45.0, 119.0), sel3(13.0, 30.0, 23.0)))
    cx = (2.0 * s[0:1] - 0.5 + x.astype(jnp.float32)) * stride
    cy = (2.0 * s[1:2] - 0.5 + y.astype(jnp.float32)) * stride
    w = (4.0 * (s[2:3] * s[2:3])) * aw
    h = (4.0 * (s[3:4] * s[3:4])) * ah
    out_ref[0, 0:1, :] = jnp.clip(cx - w * 0.5, 0.0, _IM)
    out_ref[0, 1:2, :] = jnp.clip(cy - h * 0.5, 0.0, _IM)
    out_ref[0, 2:3, :] = jnp.clip(cx + w * 0.5, 0.0, _IM)
    out_ref[0, 3:4, :] = jnp.clip(cy + h * 0.5, 0.0, _IM)


def _nms_kernel(rows_ref, fold_ref, out_ref):
    ox1 = fold_ref[:, 0]
    oy1 = fold_ref[:, 1]
    ox2 = fold_ref[:, 2]
    oy2 = fold_ref[:, 3]
    area = fold_ref[:, 4]
    sc = fold_ref[:, 5]
    sub = lax.broadcasted_iota(jnp.int32, sc.shape, 1)
    lane = lax.broadcasted_iota(jnp.int32, sc.shape, 2)
    fidx = sub * 128 + lane
    keep0 = jnp.where(sc > _SCORE_THRESH, 1.0, 0.0)

    def body(i, keepf):
        r = rows_ref[:, pl.ds(i, 1), :]
        rx1 = r[:, :, 0:1]
        ry1 = r[:, :, 1:2]
        rx2 = r[:, :, 2:3]
        ry2 = r[:, :, 3:4]
        rarea = r[:, :, 4:5]
        xx1 = jnp.maximum(rx1, ox1)
        yy1 = jnp.maximum(ry1, oy1)
        xx2 = jnp.minimum(rx2, ox2)
        yy2 = jnp.minimum(ry2, oy2)
        iw = jnp.clip(xx2 - xx1, 0.0, None)
        ih = jnp.clip(yy2 - yy1, 0.0, None)
        inter = iw * ih
        iou = inter / (rarea + area - inter + 1e-7)
        ki = jnp.max(jnp.max(jnp.where(fidx == i, keepf, 0.0), axis=2,
                             keepdims=True), axis=1, keepdims=True)
        supf = jnp.where((iou > _NMS_THRESH) & (fidx > i), 1.0, 0.0) * ki
        return keepf * (1.0 - supf)

    keepf = lax.fori_loop(0, _K_PRE, body, keep0)
    out_ref[...] = jnp.where(keepf > 0.0, sc, -1.0)


def kernel(pred0, pred1, pred2, scale_factors):
    B = pred0.shape[0]
    p0 = pred0.reshape(B, _N0, 85)
    p1 = pred1.reshape(B, _N1, 85)
    p2 = pred2.reshape(B, _N2, 85)

    s0, mb0 = _score_call(p0, 2400)
    s1, mb1 = _score_call(p1, 2400)
    s2, mb2 = _score_call(p2, 1200)

    traw = jnp.concatenate([p0[..., :4], p1[..., :4], p2[..., :4]], axis=1)
    traw = traw.transpose(0, 2, 1)
    boxes = pl.pallas_call(
        _box_kernel,
        grid=(B,),
        in_specs=[pl.BlockSpec((1, 4, _NB), lambda i: (i, 0, 0))],
        out_specs=pl.BlockSpec((1, 4, _NB), lambda i: (i, 0, 0)),
        out_shape=jax.ShapeDtypeStruct((B, 4, _NB), jnp.float32),
    )(traw)

    KB = 1536
    _, b0 = lax.top_k(mb0[:, :, 0], KB)
    _, b1 = lax.top_k(mb1[:, :, 0], KB)
    g0 = jnp.take_along_axis(s0, b0[:, :, None], axis=1)
    g1 = jnp.take_along_axis(s1, b1[:, :, None], axis=1)
    gbox = jnp.concatenate(
        [b0, b1 + _N0,
         jnp.broadcast_to(jnp.arange(_N2, dtype=b0.dtype)[None] + _N0 + _N1,
                          (B, _N2))], axis=1)
    cand = jnp.concatenate([g0, g1, s2], axis=1)
    sc, li = lax.top_k(cand.reshape(B, -1), _K_PRE)
    idx = jnp.take_along_axis(gbox, li // _NC, axis=1) * _NC + li % _NC
    bi = idx // _NC
    lab = idx % _NC
    bx4 = jnp.take_along_axis(boxes, bi[:, None, :], axis=2)
    off = lab.astype(jnp.float32) * _MAX_SIZE
    o = bx4 + off[:, None, :]
    areas = (o[:, 2] - o[:, 0]) * (o[:, 3] - o[:, 1])

    pad = _KPAD - _K_PRE
    o_p = jnp.pad(o, ((0, 0), (0, 0), (0, pad)))
    areas_p = jnp.pad(areas, ((0, 0), (0, pad)))
    sc_p = jnp.pad(sc, ((0, 0), (0, pad)), constant_values=-1.0)
    fold = jnp.concatenate(
        [o_p, areas_p[:, None, :], sc_p[:, None, :]], axis=1
    ).reshape(B, 6, 8, 128)
    rows = jnp.concatenate(
        [o_p.transpose(0, 2, 1), areas_p[..., None], sc_p[..., None],
         jnp.zeros((B, _KPAD, 2), jnp.float32)], axis=2
    )

    ks = pl.pallas_call(
        _nms_kernel,
        grid=(1,),
        in_specs=[
            pl.BlockSpec((B, _KPAD, 8), lambda i: (0, 0, 0)),
            pl.BlockSpec((B, 6, 8, 128), lambda i: (0, 0, 0, 0)),
        ],
        out_specs=pl.BlockSpec((B, 8, 128), lambda i: (0, 0, 0)),
        out_shape=jax.ShapeDtypeStruct((B, 8, 128), jnp.float32),
    )(rows, fold)

    ks = ks.reshape(B, _KPAD)[:, :_K_PRE]
    fs, fi = lax.top_k(ks, _DETECTIONS)
    fb = jnp.take_along_axis(bx4, fi[:, None, :], axis=2)
    fb = fb / scale_factors[:, None, None]
    fl = jnp.take_along_axis(lab, fi, axis=1).astype(jnp.float32)
    m = (fs > _SCORE_THRESH).astype(jnp.float32)
    fb = fb.transpose(0, 2, 1) * m[:, :, None]
    return jnp.concatenate([fb, (fs * m)[:, :, None], (fl * m)[:, :, None]], axis=2)


# speedup vs baseline: 1.4084x; 1.4084x over previous
import jax
import jax.numpy as jnp
from jax import lax
from jax.experimental import pallas as pl

_SCORE_THRESH = 0.25
_NMS_THRESH = 0.45
_DETECTIONS = 300
_K_PRE = 1000
_MAX_SIZE = 4096.0
_IM = 640.0
_NC = 80
_N0, _N1, _N2 = 19200, 4800, 1200
_NB = _N0 + _N1 + _N2
_KPAD = 1024


def _score_kernel(p_ref, out_ref, mb_ref):
    s = jax.nn.sigmoid(p_ref[0][:, 4:85])
    obj = s[:, 0:1]
    cls = s[:, 1:81]
    gate = (obj > _SCORE_THRESH).astype(jnp.float32)
    sc = (obj * cls) * gate
    sm = jnp.where(sc > _SCORE_THRESH, sc, -1.0)
    out_ref[0] = sm
    mb_ref[0] = jnp.max(sm, axis=1, keepdims=True)


def _score_call(p, chunk):
    B, N, _ = p.shape
    return pl.pallas_call(
        _score_kernel,
        grid=(B, N // chunk),
        in_specs=[pl.BlockSpec((1, chunk, 85), lambda i, j: (i, j, 0))],
        out_specs=[
            pl.BlockSpec((1, chunk, _NC), lambda i, j: (i, j, 0)),
            pl.BlockSpec((1, chunk, 1), lambda i, j: (i, j, 0)),
        ],
        out_shape=[
            jax.ShapeDtypeStruct((B, N, _NC), jnp.float32),
            jax.ShapeDtypeStruct((B, N, 1), jnp.float32),
        ],
    )(p)


def _box_kernel(t_ref, out_ref):
    s = jax.nn.sigmoid(t_ref[0])
    b = lax.broadcasted_iota(jnp.int32, (1, _NB), 1)
    l1 = b >= _N0
    l2 = b >= (_N0 + _N1)
    rel = b - jnp.where(l2, _N0 + _N1, jnp.where(l1, _N0, 0))
    y = jnp.where(l2, rel // 60, jnp.where(l1, rel // 120, rel // 240))
    q = rel // 3
    x = jnp.where(l2, q % 20, jnp.where(l1, q % 40, q % 80))
    a = rel - q * 3
    stride = jnp.where(l2, 32.0, jnp.where(l1, 16.0, 8.0))

    def sel3(v0, v1, v2):
        return jnp.where(a == 0, v0, jnp.where(a == 1, v1, v2))

    aw = jnp.where(l2, sel3(116.0, 156.0, 373.0),
                   jnp.where(l1, sel3(30.0, 62.0, 59.0), sel3(10.0, 16.0, 33.0)))
    ah = jnp.where(l2, sel3(90.0, 198.0, 326.0),
                   jnp.where(l1, sel3(61.0, 45.0, 119.0), sel3(13.0, 30.0, 23.0)))
    cx = (2.0 * s[0:1] - 0.5 + x.astype(jnp.float32)) * stride
    cy = (2.0 * s[1:2] - 0.5 + y.astype(jnp.float32)) * stride
    w = (4.0 * (s[2:3] * s[2:3])) * aw
    h = (4.0 * (s[3:4] * s[3:4])) * ah
    out_ref[0, 0:1, :] = jnp.clip(cx - w * 0.5, 0.0, _IM)
    out_ref[0, 1:2, :] = jnp.clip(cy - h * 0.5, 0.0, _IM)
    out_ref[0, 2:3, :] = jnp.clip(cx + w * 0.5, 0.0, _IM)
    out_ref[0, 3:4, :] = jnp.clip(cy + h * 0.5, 0.0, _IM)


def _nms_kernel(rows_ref, fold_ref, out_ref):
    ox1 = fold_ref[:, 0]
    oy1 = fold_ref[:, 1]
    ox2 = fold_ref[:, 2]
    oy2 = fold_ref[:, 3]
    area = fold_ref[:, 4]
    sc = fold_ref[:, 5]
    sub = lax.broadcasted_iota(jnp.int32, sc.shape, 1)
    lane = lax.broadcasted_iota(jnp.int32, sc.shape, 2)
    fidx = sub * 128 + lane
    keep0 = jnp.where(sc > _SCORE_THRESH, 1.0, 0.0)

    def body(i, keepf):
        r = rows_ref[:, pl.ds(i, 1), :]
        rx1 = r[:, :, 0:1]
        ry1 = r[:, :, 1:2]
        rx2 = r[:, :, 2:3]
        ry2 = r[:, :, 3:4]
        rarea = r[:, :, 4:5]
        xx1 = jnp.maximum(rx1, ox1)
        yy1 = jnp.maximum(ry1, oy1)
        xx2 = jnp.minimum(rx2, ox2)
        yy2 = jnp.minimum(ry2, oy2)
        iw = jnp.clip(xx2 - xx1, 0.0, None)
        ih = jnp.clip(yy2 - yy1, 0.0, None)
        inter = iw * ih
        iou = inter / (rarea + area - inter + 1e-7)
        ki = jnp.max(jnp.max(jnp.where(fidx == i, keepf, 0.0), axis=2,
                             keepdims=True), axis=1, keepdims=True)
        supf = jnp.where((iou > _NMS_THRESH) & (fidx > i), 1.0, 0.0) * ki
        return keepf * (1.0 - supf)

    keepf = lax.fori_loop(0, _K_PRE, body, keep0)
    out_ref[...] = jnp.where(keepf > 0.0, sc, -1.0)


def kernel(pred0, pred1, pred2, scale_factors):
    B = pred0.shape[0]
    p0 = pred0.reshape(B, _N0, 85)
    p1 = pred1.reshape(B, _N1, 85)
    p2 = pred2.reshape(B, _N2, 85)

    s0, mb0 = _score_call(p0, 2400)
    s1, mb1 = _score_call(p1, 2400)
    s2, mb2 = _score_call(p2, 1200)

    traw = jnp.concatenate([p0[..., :4], p1[..., :4], p2[..., :4]], axis=1)
    traw = traw.transpose(0, 2, 1)
    boxes = pl.pallas_call(
        _box_kernel,
        grid=(B,),
        in_specs=[pl.BlockSpec((1, 4, _NB), lambda i: (i, 0, 0))],
        out_specs=pl.BlockSpec((1, 4, _NB), lambda i: (i, 0, 0)),
        out_shape=jax.ShapeDtypeStruct((B, 4, _NB), jnp.float32),
    )(traw)

    KB = 1536
    _, b0 = lax.top_k(mb0[:, :, 0], KB)
    _, b1 = lax.top_k(mb1[:, :, 0], KB)
    g0 = jnp.take_along_axis(s0, b0[:, :, None], axis=1)
    g1 = jnp.take_along_axis(s1, b1[:, :, None], axis=1)
    gbox = jnp.concatenate(
        [b0, b1 + _N0,
         jnp.broadcast_to(jnp.arange(_N2, dtype=b0.dtype)[None] + _N0 + _N1,
                          (B, _N2))], axis=1)
    cand = jnp.concatenate([g0, g1, s2], axis=1)
    sc, li = lax.top_k(cand.reshape(B, -1), _K_PRE)
    idx = jnp.take_along_axis(gbox, li // _NC, axis=1) * _NC + li % _NC
    neg, idx, sc = lax.sort((-sc, idx, sc), dimension=1, num_keys=2)
    bi = idx // _NC
    lab = idx % _NC
    bx4 = jnp.take_along_axis(boxes, bi[:, None, :], axis=2)
    off = lab.astype(jnp.float32) * _MAX_SIZE
    o = bx4 + off[:, None, :]
    areas = (o[:, 2] - o[:, 0]) * (o[:, 3] - o[:, 1])

    pad = _KPAD - _K_PRE
    o_p = jnp.pad(o, ((0, 0), (0, 0), (0, pad)))
    areas_p = jnp.pad(areas, ((0, 0), (0, pad)))
    sc_p = jnp.pad(sc, ((0, 0), (0, pad)), constant_values=-1.0)
    fold = jnp.concatenate(
        [o_p, areas_p[:, None, :], sc_p[:, None, :]], axis=1
    ).reshape(B, 6, 8, 128)
    rows = jnp.concatenate(
        [o_p.transpose(0, 2, 1), areas_p[..., None], sc_p[..., None],
         jnp.zeros((B, _KPAD, 2), jnp.float32)], axis=2
    )

    ks = pl.pallas_call(
        _nms_kernel,
        grid=(1,),
        in_specs=[
            pl.BlockSpec((B, _KPAD, 8), lambda i: (0, 0, 0)),
            pl.BlockSpec((B, 6, 8, 128), lambda i: (0, 0, 0, 0)),
        ],
        out_specs=pl.BlockSpec((B, 8, 128), lambda i: (0, 0, 0)),
        out_shape=jax.ShapeDtypeStruct((B, 8, 128), jnp.float32),
    )(rows, fold)

    ks = ks.reshape(B, _KPAD)[:, :_K_PRE]
    fs, fi = lax.top_k(ks, _DETECTIONS)
    fb = jnp.take_along_axis(bx4, fi[:, None, :], axis=2)
    fb = fb / scale_factors[:, None, None]
    fl = jnp.take_along_axis(lab, fi, axis=1).astype(jnp.float32)
    m = (fs > _SCORE_THRESH).astype(jnp.float32)
    fb = fb.transpose(0, 2, 1) * m[:, :, None]
    return jnp.concatenate([fb, (fs * m)[:, :, None], (fl * m)[:, :, None]], axis=2)
